# Initial kernel scaffold; baseline (speedup 1.0000x reference)
import jax
import jax.numpy as jnp
from jax.experimental import pallas as pl


def kernel(item_sequence, sequence_lengths, table, W1, b1, g1, be1, W2, b2, g2, be2, W3, b3):
    raise NotImplementedError("write your pallas kernel here")


# speedup vs baseline: 1.0397x; 1.0397x over previous
import functools

import jax
import jax.numpy as jnp
import numpy as np
from jax import lax
from jax.experimental import pallas as pl
from jax.experimental.pallas import tpu as pltpu
from jax.experimental.pallas import tpu_sc as plsc

B = 4096
L = 200
D = 64
CH = 50
NCH = L // CH
_EPS_BN = 1e-5

_NC = 2
_NS = 16
NW = _NC * _NS
BPW = B // NW


def _pool_sc(seq, lens, table):
  mesh = plsc.VectorSubcoreMesh(core_axis_name="c", subcore_axis_name="s")

  @functools.partial(
      pl.kernel,
      out_type=jax.ShapeDtypeStruct((B, D), jnp.float32),
      mesh=mesh,
      scratch_types=[
          pltpu.VMEM((BPW, NCH, CH), jnp.int32),
          pltpu.VMEM((BPW + 16,), jnp.int32),
          pltpu.VMEM((2, NCH, CH, D), jnp.float32),
          pltpu.VMEM((BPW, D), jnp.float32),
          pltpu.SemaphoreType.DMA,
          pltpu.SemaphoreType.DMA,
      ],
      compiler_params=pltpu.CompilerParams(use_tc_tiling_on_sc=False),
  )
  def k(seq_hbm, lens_hbm, table_hbm, out_hbm, idx_v, lens_v, rows_v, out_v,
        sem0, sem1):
    sems = (sem0, sem1)
    wid = lax.axis_index("s") * _NC + lax.axis_index("c")
    base = wid * BPW
    pltpu.sync_copy(seq_hbm.at[pl.ds(base, BPW)], idx_v)
    pltpu.sync_copy(lens_hbm.at[pl.ds(base, BPW)], lens_v.at[pl.ds(0, BPW)])

    def row_len(j):
      lv = lens_v[pl.ds(j, 16)]
      return lax.min(lax.max(lv[0], 0), L)

    def fire(j, slot):
      lj = row_len(j)
      for c in range(NCH):
        @pl.when(lj > c * CH)
        def _():
          pltpu.make_async_copy(
              table_hbm.at[idx_v.at[j, c]], rows_v.at[slot, c], sems[slot]
          ).start()

    def drain(j, slot):
      lj = row_len(j)
      for c in range(NCH):
        @pl.when(lj > c * CH)
        def _():
          pltpu.make_async_copy(
              table_hbm.at[idx_v.at[j, c]], rows_v.at[slot, c], sems[slot]
          ).wait()

    def accum(j, slot):
      lj = row_len(j)
      nfull = lj // CH
      nrem = lj - nfull * CH
      zero = jnp.zeros((16,), jnp.float32)
      accs = (zero,) * 8

      def full_chunk(c, accs):
        a = list(accs[:4])
        b = list(accs[4:])
        for l in range(0, CH, 2):
          for kk in range(4):
            a[kk] = a[kk] + rows_v[slot, c, l, pl.ds(kk * 16, 16)]
            b[kk] = b[kk] + rows_v[slot, c, l + 1, pl.ds(kk * 16, 16)]
        return (*a, *b)

      accs = lax.fori_loop(0, nfull, full_chunk, accs)

      def rem_pos(l, accs):
        a = list(accs[:4])
        for kk in range(4):
          a[kk] = a[kk] + rows_v[slot, nfull, l, pl.ds(kk * 16, 16)]
        return (*a, *accs[4:])

      accs = lax.fori_loop(0, nrem, rem_pos, accs)
      for kk in range(4):
        out_v[j, pl.ds(kk * 16, 16)] = accs[kk] + accs[kk + 4]

    fire(0, 0)

    def body(g, carry):
      j0 = 2 * g
      fire(j0 + 1, 1)
      drain(j0, 0)
      accum(j0, 0)

      @pl.when(g < BPW // 2 - 1)
      def _():
        fire(j0 + 2, 0)

      drain(j0 + 1, 1)
      accum(j0 + 1, 1)
      return carry

    lax.fori_loop(0, BPW // 2, body, 0)
    pltpu.sync_copy(out_v, out_hbm.at[pl.ds(base, BPW)])

  return k(seq, lens, table)


_BB = 512


def _mlp_body(sum_ref, lens_ref, w1_ref, b1_ref, w2_ref, b2_ref, w3_ref,
              b3_ref, out_ref):
  lens = jnp.clip(lens_ref[:], 0, L).astype(jnp.float32)
  x = sum_ref[:] / (lens + 1e-9)
  h = jnp.dot(x, w1_ref[:], preferred_element_type=jnp.float32) + b1_ref[:]
  h = jnp.maximum(h, 0.0)
  h = jnp.dot(h, w2_ref[:], preferred_element_type=jnp.float32) + b2_ref[:]
  h = jnp.maximum(h, 0.0)
  o = jnp.dot(h, w3_ref[:], preferred_element_type=jnp.float32) + b3_ref[:]
  n2 = jnp.sum(o * o, axis=1, keepdims=True)
  out_ref[:] = o * lax.rsqrt(jnp.maximum(n2, 1e-24))


def _mlp_tc(psum, lens2d, w1f, b1f, w2f, b2f, w3, b3):
  h1, h2 = w1f.shape[1], w2f.shape[1]
  grid = (B // _BB,)
  return pl.pallas_call(
      _mlp_body,
      grid=grid,
      in_specs=[
          pl.BlockSpec((_BB, D), lambda i: (i, 0)),
          pl.BlockSpec((_BB, 1), lambda i: (i, 0)),
          pl.BlockSpec((D, h1), lambda i: (0, 0)),
          pl.BlockSpec((1, h1), lambda i: (0, 0)),
          pl.BlockSpec((h1, h2), lambda i: (0, 0)),
          pl.BlockSpec((1, h2), lambda i: (0, 0)),
          pl.BlockSpec((h2, D), lambda i: (0, 0)),
          pl.BlockSpec((1, D), lambda i: (0, 0)),
      ],
      out_specs=pl.BlockSpec((_BB, D), lambda i: (i, 0)),
      out_shape=jax.ShapeDtypeStruct((B, D), jnp.float32),
  )(psum, lens2d, w1f, b1f, w2f, b2f, w3, b3)


def kernel(item_sequence, sequence_lengths, table, W1, b1, g1, be1, W2, b2,
           g2, be2, W3, b3):
  seq = item_sequence.astype(jnp.int32).reshape(B, NCH, CH)
  lens = sequence_lengths.astype(jnp.int32)
  psum = _pool_sc(seq, lens, table)
  s = np.float32(1.0 / np.sqrt(1.0 + _EPS_BN))
  w1f = W1 * (g1 * s)[None, :]
  b1f = (b1 * g1 * s + be1).reshape(1, -1)
  w2f = W2 * (g2 * s)[None, :]
  b2f = (b2 * g2 * s + be2).reshape(1, -1)
  return _mlp_tc(psum, lens.reshape(B, 1), w1f, b1f, w2f, b2f, W3,
                 b3.reshape(1, -1))
